# baseline (speedup 1.0000x reference)
import functools

import jax
import jax.numpy as jnp
from jax import lax
from jax.experimental import pallas as pl
from jax.experimental.pallas import tpu as pltpu
from jax.experimental.pallas import tpu_sc as plsc

N_NODES = 10000
IN_DIM = 256
OUT_DIM = 256
N_EDGES = 160000
HALF = 128

NC = 2
NS = 16
NW = NC * NS

E_PER_TILE = N_EDGES // NW
DEG_FULL_CHUNKS = E_PER_TILE // 16
DEG_REM = E_PER_TILE - DEG_FULL_CHUNKS * 16

E_PER_SUB = N_EDGES // NS
KC = 128
NCH = 80
E_PAD_SUB = NCH * KC
PAD_ROW = N_NODES
N_ACC = N_NODES + 16
MAIN_PER_SUB = 624
WB_CH = 24
WB_N = MAIN_PER_SUB // WB_CH
TAIL_BASE = NS * MAIN_PER_SUB
TAIL_CH = N_NODES - TAIL_BASE

_mesh = plsc.VectorSubcoreMesh(
    core_axis_name="c", subcore_axis_name="s", num_cores=NC, num_subcores=NS)
_sc_params = pltpu.CompilerParams(needs_layout_passes=False)


def _deg_body(dst_hbm, deg_out, dst_v, deg_v):
    c = lax.axis_index("c")
    s = lax.axis_index("s")
    wid = s * NC + c
    zeros16 = jnp.zeros((16,), jnp.float32)
    ones16 = jnp.ones((16,), jnp.float32)

    def zero_body(i, _):
        deg_v[pl.ds(i * 16, 16)] = zeros16
        return 0
    lax.fori_loop(0, N_NODES // 16, zero_body, 0)

    dst_v[pl.ds(E_PER_TILE - 8, 16)] = jnp.zeros((16,), jnp.int32)
    pltpu.sync_copy(dst_hbm.at[pl.ds(wid * E_PER_TILE, E_PER_TILE)],
                    dst_v.at[pl.ds(0, E_PER_TILE)])

    def add_body(i, _):
        idx = dst_v[pl.ds(i * 16, 16)]
        plsc.addupdate_scatter(deg_v, [idx], ones16)
        return 0
    lax.fori_loop(0, DEG_FULL_CHUNKS, add_body, 0)

    idx = dst_v[pl.ds(DEG_FULL_CHUNKS * 16, 16)]
    mask = lax.iota(jnp.int32, 16) < DEG_REM
    plsc.addupdate_scatter(deg_v, [idx], ones16, mask=mask)

    pltpu.sync_copy(deg_v, deg_out.at[wid])


_deg_kernel = pl.kernel(
    _deg_body,
    out_type=jax.ShapeDtypeStruct((NW, N_NODES), jnp.float32),
    mesh=_mesh,
    scratch_types=[
        pltpu.VMEM((E_PER_TILE + 8,), jnp.int32),
        pltpu.VMEM((N_NODES,), jnp.float32),
    ],
    compiler_params=_sc_params,
)


def _proj_body(z_ref, w_ref, degp_ref, yl_ref, yr_ref, dis_ref):
    deg = 1.0 + jnp.sum(degp_ref[...], axis=1)
    dis = lax.rsqrt(deg)
    x = jnp.dot(z_ref[...], w_ref[...], preferred_element_type=jnp.float32)
    y = x * dis[:, None]
    yl_ref[...] = y[:, :HALF]
    yr_ref[...] = y[:, HALF:]
    dis_ref[...] = dis[:, None]


_PROJ_R = 1000

def _proj(z, w, deg_part):
    grid = N_NODES // _PROJ_R
    return pl.pallas_call(
        _proj_body,
        grid=(grid,),
        in_specs=[
            pl.BlockSpec((_PROJ_R, IN_DIM), lambda r: (r, 0)),
            pl.BlockSpec((IN_DIM, OUT_DIM), lambda r: (0, 0)),
            pl.BlockSpec((_PROJ_R, NW), lambda r: (r, 0)),
        ],
        out_specs=[
            pl.BlockSpec((_PROJ_R, HALF), lambda r: (r, 0)),
            pl.BlockSpec((_PROJ_R, HALF), lambda r: (r, 0)),
            pl.BlockSpec((_PROJ_R, 1), lambda r: (r, 0)),
        ],
        out_shape=[
            jax.ShapeDtypeStruct((N_NODES, HALF), jnp.float32),
            jax.ShapeDtypeStruct((N_NODES, HALF), jnp.float32),
            jax.ShapeDtypeStruct((N_NODES, 1), jnp.float32),
        ],
    )(z, w, deg_part)


def _agg_body(src_hbm, dst_hbm, yl_hbm, yr_hbm, dis_hbm, b_hbm, out_hbm,
              src_v, ring, rows0, rows1, dis_v, b_v, acc_s,
              sg0, sg1, si0, si1, si2, si3):
    c = lax.axis_index("c")
    s = lax.axis_index("s")

    def run(y_hbm, col0):
        pltpu.sync_copy(src_hbm.at[s], src_v)
        pltpu.async_copy(dst_hbm.at[s, 0], ring.at[0], si0)
        pltpu.async_copy(dst_hbm.at[s, 1], ring.at[1], si1)
        pltpu.async_copy(y_hbm.at[src_v.at[0]], rows0, sg0)

        pltpu.sync_copy(
            dis_hbm.at[pl.ds(pl.multiple_of(s * MAIN_PER_SUB, 8),
                             MAIN_PER_SUB)],
            dis_v.at[pl.ds(0, MAIN_PER_SUB)])
        pltpu.sync_copy(b_hbm.at[pl.ds(col0, HALF)], b_v)

        def initk(k, _):
            r0 = pl.multiple_of(s * MAIN_PER_SUB + k * WB_CH, 8)
            pltpu.sync_copy(y_hbm.at[pl.ds(r0, WB_CH)],
                            rows1.at[pl.ds(0, WB_CH)])
            pltpu.sync_copy(rows1.at[pl.ds(0, WB_CH)],
                            acc_s.at[pl.ds(r0, WB_CH)])
            return 0
        lax.fori_loop(0, WB_N, initk, 0)

        @pl.when(s == 0)
        def _():
            pltpu.sync_copy(dis_hbm.at[pl.ds(TAIL_BASE, TAIL_CH)],
                            dis_v.at[pl.ds(MAIN_PER_SUB, TAIL_CH)])
            pltpu.sync_copy(y_hbm.at[pl.ds(TAIL_BASE, TAIL_CH)],
                            rows1.at[pl.ds(0, TAIL_CH)])
            pltpu.sync_copy(rows1.at[pl.ds(0, TAIL_CH)],
                            acc_s.at[pl.ds(TAIL_BASE, TAIL_CH)])
        plsc.subcore_barrier()

        rowbufs = (rows0, rows1)
        sgs = (sg0, sg1)
        sis = (si0, si1, si2, si3)

        def quad(t, _):
            for u in range(4):
                j = 4 * t + u
                rows = rowbufs[u % 2]
                rows_n = rowbufs[(u + 1) % 2]
                slot_n2 = (u + 2) % 4

                @pl.when(j + 1 < NCH)
                def _():
                    pltpu.async_copy(
                        y_hbm.at[src_v.at[j + 1]], rows_n, sgs[(u + 1) % 2])

                pltpu.make_async_copy(
                    y_hbm.at[src_v.at[j]], rows, sgs[u % 2]).wait()

                pltpu.make_async_copy(
                    dst_hbm.at[s, j], ring.at[u], sis[u]).wait()

                @pl.when(j + 2 < NCH)
                def _():
                    pltpu.async_copy(
                        dst_hbm.at[s, j + 2], ring.at[slot_n2], sis[slot_n2])

                pltpu.sync_copy(rows, acc_s.at[ring.at[u]], add=True)
            return 0
        lax.fori_loop(0, NCH // 4, quad, 0)
        plsc.subcore_barrier()

        def scale_rows(nrows, lbase):
            def rowi(i, _):
                disb = plsc.load_gather(
                    dis_v, [jnp.zeros((16,), jnp.int32) + (lbase + i)])

                def colm(m, _):
                    v = rows1[i, pl.ds(m * 16, 16)]
                    rows1[i, pl.ds(m * 16, 16)] = v * disb + b_v[pl.ds(m * 16, 16)]
                    return 0
                lax.fori_loop(0, HALF // 16, colm, 0)
                return 0
            lax.fori_loop(0, nrows, rowi, 0)

        def wbk(k, _):
            r0 = pl.multiple_of(s * MAIN_PER_SUB + k * WB_CH, 8)
            pltpu.sync_copy(acc_s.at[pl.ds(r0, WB_CH)],
                            rows1.at[pl.ds(0, WB_CH)])
            scale_rows(WB_CH, k * WB_CH)
            pltpu.sync_copy(rows1.at[pl.ds(0, WB_CH)],
                            out_hbm.at[pl.ds(r0, WB_CH), pl.ds(col0, HALF)])
            return 0
        lax.fori_loop(0, WB_N, wbk, 0)

        @pl.when(s == 0)
        def _():
            pltpu.sync_copy(acc_s.at[pl.ds(TAIL_BASE, TAIL_CH)],
                            rows1.at[pl.ds(0, TAIL_CH)])
            scale_rows(TAIL_CH, MAIN_PER_SUB)
            pltpu.sync_copy(rows1.at[pl.ds(0, TAIL_CH)],
                            out_hbm.at[pl.ds(TAIL_BASE, TAIL_CH),
                                       pl.ds(col0, HALF)])

    @pl.when(c == 0)
    def _():
        run(yl_hbm, 0)

    @pl.when(c == 1)
    def _():
        run(yr_hbm, HALF)


_agg_kernel = pl.kernel(
    _agg_body,
    out_type=jax.ShapeDtypeStruct((N_NODES, OUT_DIM), jnp.float32),
    mesh=_mesh,
    scratch_types=[
        pltpu.VMEM((NCH, KC), jnp.int32),
        pltpu.VMEM((4, KC), jnp.int32),
        pltpu.VMEM((KC, HALF), jnp.float32),
        pltpu.VMEM((KC, HALF), jnp.float32),
        pltpu.VMEM((MAIN_PER_SUB + TAIL_CH,), jnp.float32),
        pltpu.VMEM((HALF,), jnp.float32),
        pltpu.VMEM_SHARED((N_ACC, HALF), jnp.float32),
        pltpu.SemaphoreType.DMA,
        pltpu.SemaphoreType.DMA,
        pltpu.SemaphoreType.DMA,
        pltpu.SemaphoreType.DMA,
        pltpu.SemaphoreType.DMA,
        pltpu.SemaphoreType.DMA,
    ],
    compiler_params=_sc_params,
)


@jax.jit
def kernel(z, edge_index, W, b):
    src = edge_index[0].astype(jnp.int32)
    dst = edge_index[1].astype(jnp.int32)

    deg_part = _deg_kernel(dst)
    yl, yr, dis2 = _proj(z, W, deg_part.T)

    pad_e = ((0, 0), (0, E_PAD_SUB - E_PER_SUB))
    src_p = jnp.pad(src.reshape(NS, E_PER_SUB), pad_e,
                    constant_values=PAD_ROW).reshape(NS, NCH, KC)
    dst_p = jnp.pad(dst.reshape(NS, E_PER_SUB), pad_e,
                    constant_values=PAD_ROW).reshape(NS, NCH, KC)
    pad_n = ((0, N_ACC - N_NODES), (0, 0))
    out = _agg_kernel(src_p, dst_p,
                      jnp.pad(yl, pad_n), jnp.pad(yr, pad_n),
                      dis2.reshape(N_NODES), b)
    return out


# speedup vs baseline: 1.0550x; 1.0550x over previous
import functools

import jax
import jax.numpy as jnp
from jax import lax
from jax.experimental import pallas as pl
from jax.experimental.pallas import tpu as pltpu
from jax.experimental.pallas import tpu_sc as plsc

N_NODES = 10000
IN_DIM = 256
OUT_DIM = 256
N_EDGES = 160000
HALF = 128

NC = 2
NS = 16
NW = NC * NS

E_PER_TILE = N_EDGES // NW
DEG_FULL_CHUNKS = E_PER_TILE // 16
DEG_REM = E_PER_TILE - DEG_FULL_CHUNKS * 16

E_PER_SUB = N_EDGES // NS
KC = 128
NCH = 80
E_PAD_SUB = NCH * KC
PAD_ROW = N_NODES
N_ACC = N_NODES + 16
MAIN_PER_SUB = 624
WB_CH = 24
WB_N = MAIN_PER_SUB // WB_CH
TAIL_BASE = NS * MAIN_PER_SUB
TAIL_CH = N_NODES - TAIL_BASE

_mesh = plsc.VectorSubcoreMesh(
    core_axis_name="c", subcore_axis_name="s", num_cores=NC, num_subcores=NS)
_sc_params = pltpu.CompilerParams(needs_layout_passes=False)


def _deg_body(dst_hbm, deg_out, dst_v, deg_v):
    c = lax.axis_index("c")
    s = lax.axis_index("s")
    wid = s * NC + c
    zeros16 = jnp.zeros((16,), jnp.float32)
    ones16 = jnp.ones((16,), jnp.float32)

    def zero_body(i, _):
        deg_v[pl.ds(i * 16, 16)] = zeros16
        return 0
    lax.fori_loop(0, N_NODES // 16, zero_body, 0)

    dst_v[pl.ds(E_PER_TILE - 8, 16)] = jnp.zeros((16,), jnp.int32)
    pltpu.sync_copy(dst_hbm.at[pl.ds(wid * E_PER_TILE, E_PER_TILE)],
                    dst_v.at[pl.ds(0, E_PER_TILE)])

    def add_body(i, _):
        idx = dst_v[pl.ds(i * 16, 16)]
        plsc.addupdate_scatter(deg_v, [idx], ones16)
        return 0
    lax.fori_loop(0, DEG_FULL_CHUNKS, add_body, 0)

    idx = dst_v[pl.ds(DEG_FULL_CHUNKS * 16, 16)]
    mask = lax.iota(jnp.int32, 16) < DEG_REM
    plsc.addupdate_scatter(deg_v, [idx], ones16, mask=mask)

    pltpu.sync_copy(deg_v, deg_out.at[wid])


_deg_kernel = pl.kernel(
    _deg_body,
    out_type=jax.ShapeDtypeStruct((NW, N_NODES), jnp.float32),
    mesh=_mesh,
    scratch_types=[
        pltpu.VMEM((E_PER_TILE + 8,), jnp.int32),
        pltpu.VMEM((N_NODES,), jnp.float32),
    ],
    compiler_params=_sc_params,
)


def _proj_body(z_ref, w_ref, degp_ref, yl_ref, yr_ref, dis_ref):
    deg = 1.0 + jnp.sum(degp_ref[...], axis=1)
    dis = lax.rsqrt(deg)
    x = jnp.dot(z_ref[...], w_ref[...], preferred_element_type=jnp.float32)
    y = x * dis[:, None]
    yl_ref[...] = y[:, :HALF]
    yr_ref[...] = y[:, HALF:]
    dis_ref[...] = dis[:, None]


_PROJ_R = 1000

def _proj(z, w, deg_part):
    grid = N_NODES // _PROJ_R
    return pl.pallas_call(
        _proj_body,
        grid=(grid,),
        in_specs=[
            pl.BlockSpec((_PROJ_R, IN_DIM), lambda r: (r, 0)),
            pl.BlockSpec((IN_DIM, OUT_DIM), lambda r: (0, 0)),
            pl.BlockSpec((_PROJ_R, NW), lambda r: (r, 0)),
        ],
        out_specs=[
            pl.BlockSpec((_PROJ_R, HALF), lambda r: (r, 0)),
            pl.BlockSpec((_PROJ_R, HALF), lambda r: (r, 0)),
            pl.BlockSpec((_PROJ_R, 1), lambda r: (r, 0)),
        ],
        out_shape=[
            jax.ShapeDtypeStruct((N_NODES, HALF), jnp.float32),
            jax.ShapeDtypeStruct((N_NODES, HALF), jnp.float32),
            jax.ShapeDtypeStruct((N_NODES, 1), jnp.float32),
        ],
    )(z, w, deg_part)


def _agg_body(src_hbm, dst_hbm, yl_hbm, yr_hbm, accl_out, accr_out,
              src_v, ring, rows0, rows1, acc_s,
              sg0, sg1, si0, si1, si2, si3):
    c = lax.axis_index("c")
    s = lax.axis_index("s")

    def run(y_hbm, acc_out):
        pltpu.sync_copy(src_hbm.at[s], src_v)
        pltpu.async_copy(dst_hbm.at[s, 0], ring.at[0], si0)
        pltpu.async_copy(dst_hbm.at[s, 1], ring.at[1], si1)
        pltpu.async_copy(y_hbm.at[src_v.at[0]], rows0, sg0)

        def initk(k, _):
            r0 = pl.multiple_of(s * MAIN_PER_SUB + k * WB_CH, 8)
            pltpu.sync_copy(y_hbm.at[pl.ds(r0, WB_CH)],
                            rows1.at[pl.ds(0, WB_CH)])
            pltpu.sync_copy(rows1.at[pl.ds(0, WB_CH)],
                            acc_s.at[pl.ds(r0, WB_CH)])
            return 0
        lax.fori_loop(0, WB_N, initk, 0)

        @pl.when(s == 0)
        def _():
            pltpu.sync_copy(y_hbm.at[pl.ds(TAIL_BASE, TAIL_CH)],
                            rows1.at[pl.ds(0, TAIL_CH)])
            pltpu.sync_copy(rows1.at[pl.ds(0, TAIL_CH)],
                            acc_s.at[pl.ds(TAIL_BASE, TAIL_CH)])
        plsc.subcore_barrier()

        rowbufs = (rows0, rows1)
        sgs = (sg0, sg1)
        sis = (si0, si1, si2, si3)

        def quad(t, _):
            for u in range(4):
                j = 4 * t + u
                rows = rowbufs[u % 2]
                rows_n = rowbufs[(u + 1) % 2]
                slot_n2 = (u + 2) % 4

                @pl.when(j + 1 < NCH)
                def _():
                    pltpu.async_copy(
                        y_hbm.at[src_v.at[j + 1]], rows_n, sgs[(u + 1) % 2])

                pltpu.make_async_copy(
                    y_hbm.at[src_v.at[j]], rows, sgs[u % 2]).wait()

                pltpu.make_async_copy(
                    dst_hbm.at[s, j], ring.at[u], sis[u]).wait()

                @pl.when(j + 2 < NCH)
                def _():
                    pltpu.async_copy(
                        dst_hbm.at[s, j + 2], ring.at[slot_n2], sis[slot_n2])

                pltpu.sync_copy(rows, acc_s.at[ring.at[u]], add=True)
            return 0
        lax.fori_loop(0, NCH // 4, quad, 0)
        plsc.subcore_barrier()

        def wbk(k, _):
            r0 = pl.multiple_of(s * MAIN_PER_SUB + k * WB_CH, 8)
            pltpu.sync_copy(acc_s.at[pl.ds(r0, WB_CH)],
                            rows1.at[pl.ds(0, WB_CH)])
            pltpu.sync_copy(rows1.at[pl.ds(0, WB_CH)],
                            acc_out.at[pl.ds(r0, WB_CH)])
            return 0
        lax.fori_loop(0, WB_N, wbk, 0)

        @pl.when(s == 0)
        def _():
            pltpu.sync_copy(acc_s.at[pl.ds(TAIL_BASE, TAIL_CH)],
                            rows1.at[pl.ds(0, TAIL_CH)])
            pltpu.sync_copy(rows1.at[pl.ds(0, TAIL_CH)],
                            acc_out.at[pl.ds(TAIL_BASE, TAIL_CH)])

    @pl.when(c == 0)
    def _():
        run(yl_hbm, accl_out)

    @pl.when(c == 1)
    def _():
        run(yr_hbm, accr_out)


_agg_kernel = pl.kernel(
    _agg_body,
    out_type=(
        jax.ShapeDtypeStruct((N_NODES, HALF), jnp.float32),
        jax.ShapeDtypeStruct((N_NODES, HALF), jnp.float32),
    ),
    mesh=_mesh,
    scratch_types=[
        pltpu.VMEM((NCH, KC), jnp.int32),
        pltpu.VMEM((4, KC), jnp.int32),
        pltpu.VMEM((KC, HALF), jnp.float32),
        pltpu.VMEM((KC, HALF), jnp.float32),
        pltpu.VMEM_SHARED((N_ACC, HALF), jnp.float32),
        pltpu.SemaphoreType.DMA,
        pltpu.SemaphoreType.DMA,
        pltpu.SemaphoreType.DMA,
        pltpu.SemaphoreType.DMA,
        pltpu.SemaphoreType.DMA,
        pltpu.SemaphoreType.DMA,
    ],
    compiler_params=_sc_params,
)


def _final_body(accl_ref, accr_ref, dis_ref, b_ref, out_ref):
    dis = dis_ref[...]
    out_ref[:, :HALF] = accl_ref[...] * dis + b_ref[:, :HALF]
    out_ref[:, HALF:] = accr_ref[...] * dis + b_ref[:, HALF:]


def _final(accl, accr, dis2, b2):
    grid = N_NODES // _PROJ_R
    return pl.pallas_call(
        _final_body,
        grid=(grid,),
        in_specs=[
            pl.BlockSpec((_PROJ_R, HALF), lambda r: (r, 0)),
            pl.BlockSpec((_PROJ_R, HALF), lambda r: (r, 0)),
            pl.BlockSpec((_PROJ_R, 1), lambda r: (r, 0)),
            pl.BlockSpec((1, OUT_DIM), lambda r: (0, 0)),
        ],
        out_specs=pl.BlockSpec((_PROJ_R, OUT_DIM), lambda r: (r, 0)),
        out_shape=jax.ShapeDtypeStruct((N_NODES, OUT_DIM), jnp.float32),
    )(accl, accr, dis2, b2)


@jax.jit
def kernel(z, edge_index, W, b):
    src = edge_index[0].astype(jnp.int32)
    dst = edge_index[1].astype(jnp.int32)

    deg_part = _deg_kernel(dst)
    yl, yr, dis2 = _proj(z, W, deg_part.T)

    pad_e = ((0, 0), (0, E_PAD_SUB - E_PER_SUB))
    src_p = jnp.pad(src.reshape(NS, E_PER_SUB), pad_e,
                    constant_values=PAD_ROW).reshape(NS, NCH, KC)
    dst_p = jnp.pad(dst.reshape(NS, E_PER_SUB), pad_e,
                    constant_values=PAD_ROW).reshape(NS, NCH, KC)
    pad_n = ((0, N_ACC - N_NODES), (0, 0))
    accl, accr = _agg_kernel(src_p, dst_p,
                             jnp.pad(yl, pad_n), jnp.pad(yr, pad_n))
    out = _final(accl, accr, dis2, b.reshape(1, OUT_DIM))
    return out


# speedup vs baseline: 1.0777x; 1.0215x over previous
import functools

import jax
import jax.numpy as jnp
from jax import lax
from jax.experimental import pallas as pl
from jax.experimental.pallas import tpu as pltpu
from jax.experimental.pallas import tpu_sc as plsc

N_NODES = 10000
IN_DIM = 256
OUT_DIM = 256
N_EDGES = 160000
HALF = 128

NC = 2
NS = 16
NW = NC * NS

E_PER_TILE = N_EDGES // NW
DEG_FULL_CHUNKS = E_PER_TILE // 16
DEG_REM = E_PER_TILE - DEG_FULL_CHUNKS * 16

E_PER_SUB = N_EDGES // NS
KC = 128
NCH = 80
E_PAD_SUB = NCH * KC
PAD_ROW = N_NODES
N_ACC = N_NODES + 16
MAIN_PER_SUB = 624
WB_CH = 24
WB_N = MAIN_PER_SUB // WB_CH
TAIL_BASE = NS * MAIN_PER_SUB
TAIL_CH = N_NODES - TAIL_BASE

_mesh = plsc.VectorSubcoreMesh(
    core_axis_name="c", subcore_axis_name="s", num_cores=NC, num_subcores=NS)
_sc_params = pltpu.CompilerParams(needs_layout_passes=False)


def _deg_body(dst_hbm, deg_out, dst_v, deg_v):
    c = lax.axis_index("c")
    s = lax.axis_index("s")
    wid = s * NC + c
    zeros16 = jnp.zeros((16,), jnp.float32)
    ones16 = jnp.ones((16,), jnp.float32)

    def zero_body(i, _):
        deg_v[pl.ds(i * 16, 16)] = zeros16
        return 0
    lax.fori_loop(0, N_NODES // 16, zero_body, 0)

    dst_v[pl.ds(E_PER_TILE - 8, 16)] = jnp.zeros((16,), jnp.int32)
    pltpu.sync_copy(dst_hbm.at[pl.ds(wid * E_PER_TILE, E_PER_TILE)],
                    dst_v.at[pl.ds(0, E_PER_TILE)])

    def add_body(i, _):
        idx = dst_v[pl.ds(i * 16, 16)]
        plsc.addupdate_scatter(deg_v, [idx], ones16)
        return 0
    lax.fori_loop(0, DEG_FULL_CHUNKS, add_body, 0)

    idx = dst_v[pl.ds(DEG_FULL_CHUNKS * 16, 16)]
    mask = lax.iota(jnp.int32, 16) < DEG_REM
    plsc.addupdate_scatter(deg_v, [idx], ones16, mask=mask)

    pltpu.sync_copy(deg_v, deg_out.at[wid])


_deg_kernel = pl.kernel(
    _deg_body,
    out_type=jax.ShapeDtypeStruct((NW, N_NODES), jnp.float32),
    mesh=_mesh,
    scratch_types=[
        pltpu.VMEM((E_PER_TILE + 8,), jnp.int32),
        pltpu.VMEM((N_NODES,), jnp.float32),
    ],
    compiler_params=_sc_params,
)


def _proj_body(z_ref, w_ref, degp_ref, yl_ref, yr_ref, dis_ref):
    deg = 1.0 + jnp.sum(degp_ref[...], axis=1)
    dis = lax.rsqrt(deg)
    x = jnp.dot(z_ref[...], w_ref[...], preferred_element_type=jnp.float32)
    y = x * dis[:, None]
    yl_ref[...] = y[:, :HALF]
    yr_ref[...] = y[:, HALF:]
    dis_ref[...] = dis[:, None]


_PROJ_R = 1000

def _proj(z, w, deg_part):
    grid = N_NODES // _PROJ_R
    return pl.pallas_call(
        _proj_body,
        grid=(grid,),
        in_specs=[
            pl.BlockSpec((_PROJ_R, IN_DIM), lambda r: (r, 0)),
            pl.BlockSpec((IN_DIM, OUT_DIM), lambda r: (0, 0)),
            pl.BlockSpec((_PROJ_R, NW), lambda r: (r, 0)),
        ],
        out_specs=[
            pl.BlockSpec((_PROJ_R, HALF), lambda r: (r, 0)),
            pl.BlockSpec((_PROJ_R, HALF), lambda r: (r, 0)),
            pl.BlockSpec((_PROJ_R, 1), lambda r: (r, 0)),
        ],
        out_shape=[
            jax.ShapeDtypeStruct((N_ACC, HALF), jnp.float32),
            jax.ShapeDtypeStruct((N_ACC, HALF), jnp.float32),
            jax.ShapeDtypeStruct((N_NODES, 1), jnp.float32),
        ],
    )(z, w, deg_part)


def _agg_body(src_hbm, dst_hbm, yl_hbm, yr_hbm, accl_out, accr_out,
              src_v, ring, rows0, rows1, acc_s,
              sg0, sg1, si0, si1, si2, si3):
    c = lax.axis_index("c")
    s = lax.axis_index("s")

    def run(y_hbm, acc_out):
        pltpu.sync_copy(src_hbm.at[s], src_v)
        pltpu.async_copy(dst_hbm.at[s, 0], ring.at[0], si0)
        pltpu.async_copy(dst_hbm.at[s, 1], ring.at[1], si1)
        pltpu.async_copy(y_hbm.at[src_v.at[0]], rows0, sg0)

        def initk(k, _):
            r0 = pl.multiple_of(s * MAIN_PER_SUB + k * WB_CH, 8)
            pltpu.sync_copy(y_hbm.at[pl.ds(r0, WB_CH)],
                            rows1.at[pl.ds(0, WB_CH)])
            pltpu.sync_copy(rows1.at[pl.ds(0, WB_CH)],
                            acc_s.at[pl.ds(r0, WB_CH)])
            return 0
        lax.fori_loop(0, WB_N, initk, 0)

        @pl.when(s == 0)
        def _():
            pltpu.sync_copy(y_hbm.at[pl.ds(TAIL_BASE, TAIL_CH)],
                            rows1.at[pl.ds(0, TAIL_CH)])
            pltpu.sync_copy(rows1.at[pl.ds(0, TAIL_CH)],
                            acc_s.at[pl.ds(TAIL_BASE, TAIL_CH)])
        plsc.subcore_barrier()

        rowbufs = (rows0, rows1)
        sgs = (sg0, sg1)
        sis = (si0, si1, si2, si3)

        def quad(t, _):
            for u in range(4):
                j = 4 * t + u
                rows = rowbufs[u % 2]
                rows_n = rowbufs[(u + 1) % 2]
                slot_n2 = (u + 2) % 4

                @pl.when(j + 1 < NCH)
                def _():
                    pltpu.async_copy(
                        y_hbm.at[src_v.at[j + 1]], rows_n, sgs[(u + 1) % 2])

                pltpu.make_async_copy(
                    y_hbm.at[src_v.at[j]], rows, sgs[u % 2]).wait()

                pltpu.make_async_copy(
                    dst_hbm.at[s, j], ring.at[u], sis[u]).wait()

                @pl.when(j + 2 < NCH)
                def _():
                    pltpu.async_copy(
                        dst_hbm.at[s, j + 2], ring.at[slot_n2], sis[slot_n2])

                pltpu.sync_copy(rows, acc_s.at[ring.at[u]], add=True)
            return 0
        lax.fori_loop(0, NCH // 4, quad, 0)
        plsc.subcore_barrier()

        def wbk(k, _):
            r0 = pl.multiple_of(s * MAIN_PER_SUB + k * WB_CH, 8)
            pltpu.sync_copy(acc_s.at[pl.ds(r0, WB_CH)],
                            rows1.at[pl.ds(0, WB_CH)])
            pltpu.sync_copy(rows1.at[pl.ds(0, WB_CH)],
                            acc_out.at[pl.ds(r0, WB_CH)])
            return 0
        lax.fori_loop(0, WB_N, wbk, 0)

        @pl.when(s == 0)
        def _():
            pltpu.sync_copy(acc_s.at[pl.ds(TAIL_BASE, TAIL_CH)],
                            rows1.at[pl.ds(0, TAIL_CH)])
            pltpu.sync_copy(rows1.at[pl.ds(0, TAIL_CH)],
                            acc_out.at[pl.ds(TAIL_BASE, TAIL_CH)])

    @pl.when(c == 0)
    def _():
        run(yl_hbm, accl_out)

    @pl.when(c == 1)
    def _():
        run(yr_hbm, accr_out)


_agg_kernel = pl.kernel(
    _agg_body,
    out_type=(
        jax.ShapeDtypeStruct((N_NODES, HALF), jnp.float32),
        jax.ShapeDtypeStruct((N_NODES, HALF), jnp.float32),
    ),
    mesh=_mesh,
    scratch_types=[
        pltpu.VMEM((NCH, KC), jnp.int32),
        pltpu.VMEM((4, KC), jnp.int32),
        pltpu.VMEM((KC, HALF), jnp.float32),
        pltpu.VMEM((KC, HALF), jnp.float32),
        pltpu.VMEM_SHARED((N_ACC, HALF), jnp.float32),
        pltpu.SemaphoreType.DMA,
        pltpu.SemaphoreType.DMA,
        pltpu.SemaphoreType.DMA,
        pltpu.SemaphoreType.DMA,
        pltpu.SemaphoreType.DMA,
        pltpu.SemaphoreType.DMA,
    ],
    compiler_params=_sc_params,
)


def _final_body(accl_ref, accr_ref, dis_ref, b_ref, out_ref):
    dis = dis_ref[...]
    out_ref[:, :HALF] = accl_ref[...] * dis + b_ref[:, :HALF]
    out_ref[:, HALF:] = accr_ref[...] * dis + b_ref[:, HALF:]


def _final(accl, accr, dis2, b2):
    grid = N_NODES // _PROJ_R
    return pl.pallas_call(
        _final_body,
        grid=(grid,),
        in_specs=[
            pl.BlockSpec((_PROJ_R, HALF), lambda r: (r, 0)),
            pl.BlockSpec((_PROJ_R, HALF), lambda r: (r, 0)),
            pl.BlockSpec((_PROJ_R, 1), lambda r: (r, 0)),
            pl.BlockSpec((1, OUT_DIM), lambda r: (0, 0)),
        ],
        out_specs=pl.BlockSpec((_PROJ_R, OUT_DIM), lambda r: (r, 0)),
        out_shape=jax.ShapeDtypeStruct((N_NODES, OUT_DIM), jnp.float32),
    )(accl, accr, dis2, b2)


@jax.jit
def kernel(z, edge_index, W, b):
    src = edge_index[0].astype(jnp.int32)
    dst = edge_index[1].astype(jnp.int32)

    deg_part = _deg_kernel(dst)
    yl, yr, dis2 = _proj(z, W, deg_part.T)

    pad_e = ((0, 0), (0, E_PAD_SUB - E_PER_SUB))
    src_p = jnp.pad(src.reshape(NS, E_PER_SUB), pad_e,
                    constant_values=PAD_ROW).reshape(NS, NCH, KC)
    dst_p = jnp.pad(dst.reshape(NS, E_PER_SUB), pad_e,
                    constant_values=PAD_ROW).reshape(NS, NCH, KC)
    accl, accr = _agg_kernel(src_p, dst_p, yl, yr)
    out = _final(accl, accr, dis2, b.reshape(1, OUT_DIM))
    return out


# speedup vs baseline: 1.0786x; 1.0009x over previous
import jax
import jax.numpy as jnp
from jax import lax
from jax.experimental import pallas as pl
from jax.experimental.pallas import tpu as pltpu
from jax.experimental.pallas import tpu_sc as plsc

N_NODES = 10000
IN_DIM = 256
OUT_DIM = 256
N_EDGES = 160000
HALF = 128

NC = 2
NS = 16
NW = NC * NS

E_PER_TILE = N_EDGES // NW
DEG_FULL_CHUNKS = E_PER_TILE // 16
DEG_REM = E_PER_TILE - DEG_FULL_CHUNKS * 16

E_PER_SUB = N_EDGES // NS
KC = 128
NCH = 80
E_PAD_SUB = NCH * KC
PAD_ROW = N_NODES
N_ACC = N_NODES + 16
MAIN_PER_SUB = 624
WB_CH = 24
WB_N = MAIN_PER_SUB // WB_CH
TAIL_BASE = NS * MAIN_PER_SUB
TAIL_CH = N_NODES - TAIL_BASE

_mesh = plsc.VectorSubcoreMesh(
    core_axis_name="c", subcore_axis_name="s", num_cores=NC, num_subcores=NS)
_sc_params = pltpu.CompilerParams(needs_layout_passes=False)


def _deg_body(dst_hbm, deg_out, dst_v, deg_v):
    c = lax.axis_index("c")
    s = lax.axis_index("s")
    wid = s * NC + c
    zeros16 = jnp.zeros((16,), jnp.float32)
    ones16 = jnp.ones((16,), jnp.float32)

    def zero_body(i, _):
        deg_v[pl.ds(i * 16, 16)] = zeros16
        return 0
    lax.fori_loop(0, N_NODES // 16, zero_body, 0)

    dst_v[pl.ds(E_PER_TILE - 8, 16)] = jnp.zeros((16,), jnp.int32)
    pltpu.sync_copy(dst_hbm.at[pl.ds(wid * E_PER_TILE, E_PER_TILE)],
                    dst_v.at[pl.ds(0, E_PER_TILE)])

    def add_body(i, _):
        idx = dst_v[pl.ds(i * 16, 16)]
        plsc.addupdate_scatter(deg_v, [idx], ones16)
        return 0
    lax.fori_loop(0, DEG_FULL_CHUNKS, add_body, 0)

    idx = dst_v[pl.ds(DEG_FULL_CHUNKS * 16, 16)]
    mask = lax.iota(jnp.int32, 16) < DEG_REM
    plsc.addupdate_scatter(deg_v, [idx], ones16, mask=mask)

    pltpu.sync_copy(deg_v, deg_out.at[wid])


_deg_kernel = pl.kernel(
    _deg_body,
    out_type=jax.ShapeDtypeStruct((NW, N_NODES), jnp.float32),
    mesh=_mesh,
    scratch_types=[
        pltpu.VMEM((E_PER_TILE + 8,), jnp.int32),
        pltpu.VMEM((N_NODES,), jnp.float32),
    ],
    compiler_params=_sc_params,
)


def _proj_body(z_ref, w_ref, degp_ref, yl_ref, yr_ref, dis_ref):
    deg = 1.0 + jnp.sum(degp_ref[...], axis=1)
    dis = lax.rsqrt(deg)
    x = jnp.dot(z_ref[...], w_ref[...], preferred_element_type=jnp.float32)
    y = x * dis[:, None]
    yl_ref[...] = y[:, :HALF]
    yr_ref[...] = y[:, HALF:]
    dis_ref[...] = dis[:, None]


_PROJ_R = 1000

def _proj(z, w, deg_part):
    grid = N_NODES // _PROJ_R
    return pl.pallas_call(
        _proj_body,
        grid=(grid,),
        in_specs=[
            pl.BlockSpec((_PROJ_R, IN_DIM), lambda r: (r, 0)),
            pl.BlockSpec((IN_DIM, OUT_DIM), lambda r: (0, 0)),
            pl.BlockSpec((_PROJ_R, NW), lambda r: (r, 0)),
        ],
        out_specs=[
            pl.BlockSpec((_PROJ_R, HALF), lambda r: (r, 0)),
            pl.BlockSpec((_PROJ_R, HALF), lambda r: (r, 0)),
            pl.BlockSpec((_PROJ_R, 1), lambda r: (r, 0)),
        ],
        out_shape=[
            jax.ShapeDtypeStruct((N_ACC, HALF), jnp.float32),
            jax.ShapeDtypeStruct((N_ACC, HALF), jnp.float32),
            jax.ShapeDtypeStruct((N_NODES, 1), jnp.float32),
        ],
    )(z, w, deg_part)


def _agg_body(src_hbm, dst_hbm, yl_hbm, yr_hbm, accl_out, accr_out,
              src_v, ring, rows0, rows1, acc_s,
              sg0, sg1, si0, si1, si2, si3):
    c = lax.axis_index("c")
    s = lax.axis_index("s")

    def run(y_hbm, acc_out):
        pltpu.sync_copy(src_hbm.at[s], src_v)
        pltpu.async_copy(dst_hbm.at[s, 0], ring.at[0], si0)
        pltpu.async_copy(dst_hbm.at[s, 1], ring.at[1], si1)
        pltpu.async_copy(y_hbm.at[src_v.at[0]], rows0, sg0)

        def initk(k, _):
            r0 = pl.multiple_of(s * MAIN_PER_SUB + k * WB_CH, 8)
            pltpu.sync_copy(y_hbm.at[pl.ds(r0, WB_CH)],
                            rows1.at[pl.ds(0, WB_CH)])
            pltpu.sync_copy(rows1.at[pl.ds(0, WB_CH)],
                            acc_s.at[pl.ds(r0, WB_CH)])
            return 0
        lax.fori_loop(0, WB_N, initk, 0)

        @pl.when(s == 0)
        def _():
            pltpu.sync_copy(y_hbm.at[pl.ds(TAIL_BASE, TAIL_CH)],
                            rows1.at[pl.ds(0, TAIL_CH)])
            pltpu.sync_copy(rows1.at[pl.ds(0, TAIL_CH)],
                            acc_s.at[pl.ds(TAIL_BASE, TAIL_CH)])
        plsc.subcore_barrier()

        rowbufs = (rows0, rows1)
        sgs = (sg0, sg1)
        sis = (si0, si1, si2, si3)

        def quad(t, _):
            for u in range(4):
                j = 4 * t + u
                rows = rowbufs[u % 2]
                rows_n = rowbufs[(u + 1) % 2]
                slot_n2 = (u + 2) % 4

                @pl.when(j + 1 < NCH)
                def _():
                    pltpu.async_copy(
                        y_hbm.at[src_v.at[j + 1]], rows_n, sgs[(u + 1) % 2])

                pltpu.make_async_copy(
                    y_hbm.at[src_v.at[j]], rows, sgs[u % 2]).wait()

                pltpu.make_async_copy(
                    dst_hbm.at[s, j], ring.at[u], sis[u]).wait()

                @pl.when(j + 2 < NCH)
                def _():
                    pltpu.async_copy(
                        dst_hbm.at[s, j + 2], ring.at[slot_n2], sis[slot_n2])

                pltpu.sync_copy(rows, acc_s.at[ring.at[u]], add=True)
            return 0
        lax.fori_loop(0, NCH // 4, quad, 0)
        plsc.subcore_barrier()

        def wbk(k, _):
            r0 = pl.multiple_of(s * MAIN_PER_SUB + k * WB_CH, 8)
            pltpu.sync_copy(acc_s.at[pl.ds(r0, WB_CH)],
                            rows1.at[pl.ds(0, WB_CH)])
            pltpu.sync_copy(rows1.at[pl.ds(0, WB_CH)],
                            acc_out.at[pl.ds(r0, WB_CH)])
            return 0
        lax.fori_loop(0, WB_N, wbk, 0)

        @pl.when(s == 0)
        def _():
            pltpu.sync_copy(acc_s.at[pl.ds(TAIL_BASE, TAIL_CH)],
                            rows1.at[pl.ds(0, TAIL_CH)])
            pltpu.sync_copy(rows1.at[pl.ds(0, TAIL_CH)],
                            acc_out.at[pl.ds(TAIL_BASE, TAIL_CH)])

    @pl.when(c == 0)
    def _():
        run(yl_hbm, accl_out)

    @pl.when(c == 1)
    def _():
        run(yr_hbm, accr_out)


_agg_kernel = pl.kernel(
    _agg_body,
    out_type=(
        jax.ShapeDtypeStruct((N_NODES, HALF), jnp.float32),
        jax.ShapeDtypeStruct((N_NODES, HALF), jnp.float32),
    ),
    mesh=_mesh,
    scratch_types=[
        pltpu.VMEM((NCH, KC), jnp.int32),
        pltpu.VMEM((4, KC), jnp.int32),
        pltpu.VMEM((KC, HALF), jnp.float32),
        pltpu.VMEM((KC, HALF), jnp.float32),
        pltpu.VMEM_SHARED((N_ACC, HALF), jnp.float32),
        pltpu.SemaphoreType.DMA,
        pltpu.SemaphoreType.DMA,
        pltpu.SemaphoreType.DMA,
        pltpu.SemaphoreType.DMA,
        pltpu.SemaphoreType.DMA,
        pltpu.SemaphoreType.DMA,
    ],
    compiler_params=_sc_params,
)


def _final_body(accl_ref, accr_ref, dis_ref, b_ref, out_ref):
    dis = dis_ref[...]
    out_ref[:, :HALF] = accl_ref[...] * dis + b_ref[:, :HALF]
    out_ref[:, HALF:] = accr_ref[...] * dis + b_ref[:, HALF:]


def _final(accl, accr, dis2, b2):
    grid = N_NODES // _PROJ_R
    return pl.pallas_call(
        _final_body,
        grid=(grid,),
        in_specs=[
            pl.BlockSpec((_PROJ_R, HALF), lambda r: (r, 0)),
            pl.BlockSpec((_PROJ_R, HALF), lambda r: (r, 0)),
            pl.BlockSpec((_PROJ_R, 1), lambda r: (r, 0)),
            pl.BlockSpec((1, OUT_DIM), lambda r: (0, 0)),
        ],
        out_specs=pl.BlockSpec((_PROJ_R, OUT_DIM), lambda r: (r, 0)),
        out_shape=jax.ShapeDtypeStruct((N_NODES, OUT_DIM), jnp.float32),
    )(accl, accr, dis2, b2)


@jax.jit
def kernel(z, edge_index, W, b):
    src = edge_index[0].astype(jnp.int32)
    dst = edge_index[1].astype(jnp.int32)

    deg_part = _deg_kernel(dst)
    yl, yr, dis2 = _proj(z, W, deg_part.T)

    pad_e = ((0, 0), (0, E_PAD_SUB - E_PER_SUB))
    src_p = jnp.pad(src.reshape(NS, E_PER_SUB), pad_e,
                    constant_values=PAD_ROW).reshape(NS, NCH, KC)
    dst_p = jnp.pad(dst.reshape(NS, E_PER_SUB), pad_e,
                    constant_values=PAD_ROW).reshape(NS, NCH, KC)
    accl, accr = _agg_kernel(src_p, dst_p, yl, yr)
    out = _final(accl, accr, dis2, b.reshape(1, OUT_DIM))
    return out
